# speedup vs baseline: 1.0202x; 1.0202x over previous
import functools

import jax
import jax.numpy as jnp
from jax import lax
from jax.experimental import pallas as pl
from jax.experimental.pallas import tpu as pltpu
from jax.experimental.pallas import tpu_sc as plsc

N, T, E = 64, 512, 768
D, G = 8, 8
DG = D * G
NC, NS = 2, 16
NW = NC * NS
TPW = T // NW
L = 16
EC = E // L


def _pooler_body(p_hbm, idx_hbm, out_hbm, base_v, idx_all_v, rows_v, out_v, sem):
    cid = lax.axis_index("c")
    sid = lax.axis_index("s")
    wid = sid * NC + cid
    t0 = wid * TPW

    pltpu.sync_copy(idx_hbm, base_v)
    for tl in range(TPW):
        t = t0 + tl
        for k in range(DG // L):
            sl = pl.ds(k * L, L)
            idx_all_v[tl, sl] = base_v[sl] + t

    def t_body(tl, carry):
        pltpu.async_copy(p_hbm.at[idx_all_v.at[tl]], rows_v, sem).wait()

        def d_body(d, carry2):
            r0 = d * G
            for c in range(EC):
                sl = pl.ds(c * L, L)
                m = rows_v[r0, sl]
                for g in range(1, G):
                    m = jnp.maximum(m, rows_v[r0 + g, sl])
                out_v[d, sl] = m
            return carry2

        lax.fori_loop(0, D, d_body, 0)
        pltpu.sync_copy(out_v, out_hbm.at[t0 + tl])
        return carry

    lax.fori_loop(0, TPW, t_body, 0)


@jax.jit
def _pooler(p_flat, idx_scaled):
    mesh = plsc.VectorSubcoreMesh(core_axis_name="c", subcore_axis_name="s")
    f = functools.partial(
        pl.kernel,
        out_type=jax.ShapeDtypeStruct((T, D, E), jnp.float32),
        mesh=mesh,
        scratch_types=[
            pltpu.VMEM((DG,), jnp.int32),
            pltpu.VMEM((TPW, DG), jnp.int32),
            pltpu.VMEM((DG, E), jnp.float32),
            pltpu.VMEM((D, E), jnp.float32),
            pltpu.SemaphoreType.DMA,
        ],
    )(_pooler_body)
    return f(p_flat, idx_scaled)


def kernel(P, subgroup_indices):
    p_flat = P.reshape(N * T, E)
    idx_scaled = (subgroup_indices.astype(jnp.int32) * jnp.int32(T)).reshape(DG)
    out = _pooler(p_flat, idx_scaled)
    return out.reshape(1, T, D * E)


# speedup vs baseline: 1.4021x; 1.3743x over previous
import functools

import jax
import jax.numpy as jnp
from jax import lax
from jax.experimental import pallas as pl
from jax.experimental.pallas import tpu as pltpu
from jax.experimental.pallas import tpu_sc as plsc

N, T, E = 64, 512, 768
D, G = 8, 8
DG = D * G
NC, NS = 2, 16
NW = NC * NS
TPW = T // NW
L = 16
EC = E // L


def _pooler_body(p_hbm, idx_hbm, out_hbm, base_v, idx_all_v, rows_v, out_v,
                 gsem, osem):
    cid = lax.axis_index("c")
    sid = lax.axis_index("s")
    wid = sid * NC + cid
    t0 = wid * TPW

    pltpu.sync_copy(idx_hbm, base_v)
    for tl in range(TPW):
        t = t0 + tl
        for k in range(DG // L):
            sl = pl.ds(k * L, L)
            idx_all_v[tl, sl] = base_v[sl] + t

    def gather_start(tl, b):
        pltpu.async_copy(p_hbm.at[idx_all_v.at[tl]], rows_v.at[b], gsem)

    def gather_wait(b):
        pltpu.make_async_copy(p_hbm.at[idx_all_v.at[0]], rows_v.at[b],
                              gsem).wait()

    gather_start(0, 0)
    gather_start(1, 1)

    def t2_body(tlo, carry):
        for b in range(2):
            tl = tlo + b
            gather_wait(b)
            @pl.when(tl >= 2)
            def _():
                pltpu.make_async_copy(out_v.at[b], out_hbm.at[t0], osem).wait()

            def d_body(d, carry2):
                r0 = d * G
                for c in range(EC):
                    sl = pl.ds(c * L, L)
                    m = jnp.maximum(
                        jnp.maximum(
                            jnp.maximum(rows_v[b, r0, sl], rows_v[b, r0 + 1, sl]),
                            jnp.maximum(rows_v[b, r0 + 2, sl], rows_v[b, r0 + 3, sl]),
                        ),
                        jnp.maximum(
                            jnp.maximum(rows_v[b, r0 + 4, sl], rows_v[b, r0 + 5, sl]),
                            jnp.maximum(rows_v[b, r0 + 6, sl], rows_v[b, r0 + 7, sl]),
                        ),
                    )
                    out_v[b, d, sl] = m
                return carry2

            lax.fori_loop(0, D, d_body, 0)
            pltpu.async_copy(out_v.at[b], out_hbm.at[t0 + tl], osem)

            @pl.when(tl + 2 < TPW)
            def _():
                gather_start(tl + 2, b)
        return carry

    lax.fori_loop(0, TPW // 2, lambda i, c: t2_body(i * 2, c), 0)

    for b in range(2):
        pltpu.make_async_copy(out_v.at[b], out_hbm.at[t0], osem).wait()


@jax.jit
def _pooler(p_flat, idx_scaled):
    mesh = plsc.VectorSubcoreMesh(core_axis_name="c", subcore_axis_name="s")
    f = functools.partial(
        pl.kernel,
        out_type=jax.ShapeDtypeStruct((T, D, E), jnp.float32),
        mesh=mesh,
        scratch_types=[
            pltpu.VMEM((DG,), jnp.int32),
            pltpu.VMEM((TPW, DG), jnp.int32),
            pltpu.VMEM((2, DG, E), jnp.float32),
            pltpu.VMEM((2, D, E), jnp.float32),
            pltpu.SemaphoreType.DMA,
            pltpu.SemaphoreType.DMA,
        ],
    )(_pooler_body)
    return f(p_flat, idx_scaled)


def kernel(P, subgroup_indices):
    p_flat = P.reshape(N * T, E)
    idx_scaled = (subgroup_indices.astype(jnp.int32) * jnp.int32(T)).reshape(DG)
    out = _pooler(p_flat, idx_scaled)
    return out.reshape(1, T, D * E)
